# baseline (speedup 1.0000x reference)
import jax
import jax.numpy as jnp
from jax import lax
from jax.experimental import pallas as pl
from jax.experimental.pallas import tpu as pltpu

_N = 10000
_NIN = 128
_NH = 64
_BM = 200
_NB = _N // _BM


def _body(adj_ref, seq1_ref, seq2_ref, w_ref, b_ref, a_ref, mskc_ref,
          wbil_ref, bbil_ref, sc_ref, fts_ref, g_scr, red_ref):
    i = pl.program_id(0)

    @pl.when(i == 0)
    def _prologue():
        w = w_ref[...]
        dn = (((1,), (1,)), ((), ()))
        fts_ref[:, :_NH] = lax.dot_general(
            seq1_ref[...], w, dn, preferred_element_type=jnp.float32)
        fts_ref[:, _NH:] = lax.dot_general(
            seq2_ref[...], w, dn, preferred_element_type=jnp.float32)
        red_ref[...] = jnp.zeros((1, 2 * _NH), jnp.float32)

    out = jnp.dot(adj_ref[...], fts_ref[...], preferred_element_type=jnp.float32)
    b = b_ref[...]
    a = a_ref[0, 0]
    o1 = out[:, :_NH] + b
    o2 = out[:, _NH:] + b
    h1 = jnp.where(o1 > 0, o1, a * o1)
    h2 = jnp.where(o2 > 0, o2, a * o2)
    wb = wbil_ref[...]
    g_scr[pl.ds(i * _BM, _BM), :_NH] = jnp.dot(
        h1, wb, preferred_element_type=jnp.float32)
    g_scr[pl.ds(i * _BM, _BM), _NH:] = jnp.dot(
        h2, wb, preferred_element_type=jnp.float32)
    mskc = mskc_ref[...]
    red_ref[:, :_NH] += jnp.sum(h1 * mskc, axis=0, keepdims=True)
    red_ref[:, _NH:] += jnp.sum(
        jnp.broadcast_to(mskc, (_BM, _NH)), axis=0, keepdims=True)

    @pl.when(i == _NB - 1)
    def _epilogue():
        red = red_ref[...]
        c = jax.nn.sigmoid(red[0:1, :_NH] / red[0, _NH])
        bb = bbil_ref[0, 0]
        sc_ref[:, 0:1] = jnp.sum(g_scr[:, :_NH] * c, axis=1, keepdims=True) + bb
        sc_ref[:, 1:2] = jnp.sum(g_scr[:, _NH:] * c, axis=1, keepdims=True) + bb


def kernel(seq1, seq2, adj, sparse, msk, samp_bias1, samp_bias2,
           W_fc, b_gcn, prelu_a, W_bil, b_bil):
    del sparse
    seq1_2d = seq1.reshape(_N, _NIN)
    seq2_2d = seq2.reshape(_N, _NIN)
    adj_2d = adj.reshape(_N, _N)
    b1r = b_gcn.reshape(1, _NH)
    a11 = prelu_a.reshape(1, 1)
    bbil11 = b_bil.reshape(1, 1)
    mskc = msk.reshape(_N, 1)

    pin = lambda i: (0, 0)
    sc = pl.pallas_call(
        _body,
        grid=(_NB,),
        in_specs=[
            pl.BlockSpec((_BM, _N), lambda i: (i, 0)),
            pl.BlockSpec((_N, _NIN), pin),
            pl.BlockSpec((_N, _NIN), pin),
            pl.BlockSpec((_NH, _NIN), pin),
            pl.BlockSpec((1, _NH), pin),
            pl.BlockSpec((1, 1), pin),
            pl.BlockSpec((_BM, 1), lambda i: (i, 0)),
            pl.BlockSpec((_NH, _NH), pin),
            pl.BlockSpec((1, 1), pin),
        ],
        out_specs=pl.BlockSpec((_N, 2), pin),
        out_shape=jax.ShapeDtypeStruct((_N, 2), jnp.float32),
        scratch_shapes=[
            pltpu.VMEM((_N, 2 * _NH), jnp.float32),
            pltpu.VMEM((_N, 2 * _NH), jnp.float32),
            pltpu.VMEM((1, 2 * _NH), jnp.float32),
        ],
    )(adj_2d, seq1_2d, seq2_2d, W_fc, b1r, a11, mskc,
      W_bil.reshape(_NH, _NH), bbil11)

    logits = sc.T.reshape(1, 2 * _N)
    return logits + jnp.concatenate([samp_bias1, samp_bias2], axis=1)


# speedup vs baseline: 1.1063x; 1.1063x over previous
import jax
import jax.numpy as jnp
from jax import lax
from jax.experimental import pallas as pl
from jax.experimental.pallas import tpu as pltpu

_N = 10000
_NIN = 128
_NH = 64
_BM = 200
_NB = _N // _BM


def _body(adj_ref, seq1_ref, seq2_ref, w_ref, b_ref, a_ref, mskr_ref,
          wbil_ref, bbil_ref, sc_ref, fts_ref, h_scr):
    i = pl.program_id(0)

    @pl.when(i == 0)
    def _prologue():
        w = w_ref[...]
        dn = (((1,), (1,)), ((), ()))
        fts_ref[:, :_NH] = lax.dot_general(
            seq1_ref[...], w, dn, preferred_element_type=jnp.float32)
        fts_ref[:, _NH:] = lax.dot_general(
            seq2_ref[...], w, dn, preferred_element_type=jnp.float32)

    out = jnp.dot(adj_ref[...], fts_ref[...], preferred_element_type=jnp.float32)
    b = b_ref[...]
    a = a_ref[0, 0]
    o1 = out[:, :_NH] + b
    o2 = out[:, _NH:] + b
    h_scr[pl.ds(i * _BM, _BM), :_NH] = jnp.where(o1 > 0, o1, a * o1)
    h_scr[pl.ds(i * _BM, _BM), _NH:] = jnp.where(o2 > 0, o2, a * o2)

    @pl.when(i == _NB - 1)
    def _epilogue():
        hf = h_scr[...]
        mskr = mskr_ref[...]
        mskb = jnp.broadcast_to(mskr, (8, _N))
        csum = lax.dot_general(mskb, hf[:, :_NH], (((1,), (0,)), ((), ())),
                               preferred_element_type=jnp.float32)
        c = jax.nn.sigmoid(csum[0:1, :] / jnp.sum(mskr))
        rows = lax.broadcasted_iota(jnp.int32, (_NH, _NH), 0)
        cols = lax.broadcasted_iota(jnp.int32, (_NH, _NH), 1)
        cdiag = jnp.where(rows == cols, jnp.broadcast_to(c, (_NH, _NH)), 0.0)
        ccol = jnp.sum(cdiag, axis=1, keepdims=True)
        v = jnp.dot(wbil_ref[...], ccol,
                    preferred_element_type=jnp.float32)
        z = jnp.zeros((_NH, 1), jnp.float32)
        v2 = jnp.concatenate(
            [jnp.concatenate([v, z], axis=0),
             jnp.concatenate([z, v], axis=0)], axis=1)
        bb = bbil_ref[0, 0]
        sc_ref[...] = jnp.dot(hf, v2, preferred_element_type=jnp.float32) + bb


def kernel(seq1, seq2, adj, sparse, msk, samp_bias1, samp_bias2,
           W_fc, b_gcn, prelu_a, W_bil, b_bil):
    del sparse
    seq1_2d = seq1.reshape(_N, _NIN)
    seq2_2d = seq2.reshape(_N, _NIN)
    adj_2d = adj.reshape(_N, _N)
    b1r = b_gcn.reshape(1, _NH)
    a11 = prelu_a.reshape(1, 1)
    bbil11 = b_bil.reshape(1, 1)

    pin = lambda i: (0, 0)
    sc = pl.pallas_call(
        _body,
        grid=(_NB,),
        in_specs=[
            pl.BlockSpec((_BM, _N), lambda i: (i, 0)),
            pl.BlockSpec((_N, _NIN), pin),
            pl.BlockSpec((_N, _NIN), pin),
            pl.BlockSpec((_NH, _NIN), pin),
            pl.BlockSpec((1, _NH), pin),
            pl.BlockSpec((1, 1), pin),
            pl.BlockSpec((1, _N), pin),
            pl.BlockSpec((_NH, _NH), pin),
            pl.BlockSpec((1, 1), pin),
        ],
        out_specs=pl.BlockSpec((_N, 2), pin),
        out_shape=jax.ShapeDtypeStruct((_N, 2), jnp.float32),
        scratch_shapes=[
            pltpu.VMEM((_N, 2 * _NH), jnp.float32),
            pltpu.VMEM((_N, 2 * _NH), jnp.float32),
        ],
    )(adj_2d, seq1_2d, seq2_2d, W_fc, b1r, a11, msk,
      W_bil.reshape(_NH, _NH), bbil11)

    logits = sc.T.reshape(1, 2 * _N)
    return logits + jnp.concatenate([samp_bias1, samp_bias2], axis=1)


# speedup vs baseline: 1.1202x; 1.0125x over previous
import jax
import jax.numpy as jnp
from jax import lax
from jax.experimental import pallas as pl
from jax.experimental.pallas import tpu as pltpu

_N = 10000
_NIN = 128
_NH = 64
_BM = 400
_NB = _N // _BM


def _body(adj_ref, seq1_ref, seq2_ref, w_ref, b_ref, a_ref, mskr_ref,
          wbil_ref, bbil_ref, sc_ref, fts_ref, h_scr):
    i = pl.program_id(0)

    @pl.when(i == 0)
    def _prologue():
        w = w_ref[...]
        dn = (((1,), (1,)), ((), ()))
        fts_ref[:, :_NH] = lax.dot_general(
            seq1_ref[...], w, dn, preferred_element_type=jnp.float32)
        fts_ref[:, _NH:] = lax.dot_general(
            seq2_ref[...], w, dn, preferred_element_type=jnp.float32)

    out = jnp.dot(adj_ref[...], fts_ref[...], preferred_element_type=jnp.float32)
    b = b_ref[...]
    a = a_ref[0, 0]
    o1 = out[:, :_NH] + b
    o2 = out[:, _NH:] + b
    h_scr[pl.ds(i * _BM, _BM), :_NH] = jnp.where(o1 > 0, o1, a * o1)
    h_scr[pl.ds(i * _BM, _BM), _NH:] = jnp.where(o2 > 0, o2, a * o2)

    @pl.when(i == _NB - 1)
    def _epilogue():
        hf = h_scr[...]
        mskr = mskr_ref[...]
        mskb = jnp.broadcast_to(mskr, (8, _N))
        csum = lax.dot_general(mskb, hf[:, :_NH], (((1,), (0,)), ((), ())),
                               preferred_element_type=jnp.float32)
        c = jax.nn.sigmoid(csum[0:1, :] / jnp.sum(mskr))
        rows = lax.broadcasted_iota(jnp.int32, (_NH, _NH), 0)
        cols = lax.broadcasted_iota(jnp.int32, (_NH, _NH), 1)
        cdiag = jnp.where(rows == cols, jnp.broadcast_to(c, (_NH, _NH)), 0.0)
        ccol = jnp.sum(cdiag, axis=1, keepdims=True)
        v = jnp.dot(wbil_ref[...], ccol,
                    preferred_element_type=jnp.float32)
        z = jnp.zeros((_NH, 1), jnp.float32)
        v2 = jnp.concatenate(
            [jnp.concatenate([v, z], axis=0),
             jnp.concatenate([z, v], axis=0)], axis=1)
        bb = bbil_ref[0, 0]
        sc_ref[...] = jnp.dot(hf, v2, preferred_element_type=jnp.float32) + bb


def kernel(seq1, seq2, adj, sparse, msk, samp_bias1, samp_bias2,
           W_fc, b_gcn, prelu_a, W_bil, b_bil):
    del sparse
    seq1_2d = seq1.reshape(_N, _NIN)
    seq2_2d = seq2.reshape(_N, _NIN)
    adj_2d = adj.reshape(_N, _N)
    b1r = b_gcn.reshape(1, _NH)
    a11 = prelu_a.reshape(1, 1)
    bbil11 = b_bil.reshape(1, 1)

    pin = lambda i: (0, 0)
    sc = pl.pallas_call(
        _body,
        grid=(_NB,),
        in_specs=[
            pl.BlockSpec((_BM, _N), lambda i: (i, 0)),
            pl.BlockSpec((_N, _NIN), pin),
            pl.BlockSpec((_N, _NIN), pin),
            pl.BlockSpec((_NH, _NIN), pin),
            pl.BlockSpec((1, _NH), pin),
            pl.BlockSpec((1, 1), pin),
            pl.BlockSpec((1, _N), pin),
            pl.BlockSpec((_NH, _NH), pin),
            pl.BlockSpec((1, 1), pin),
        ],
        out_specs=pl.BlockSpec((_N, 2), pin),
        out_shape=jax.ShapeDtypeStruct((_N, 2), jnp.float32),
        scratch_shapes=[
            pltpu.VMEM((_N, 2 * _NH), jnp.float32),
            pltpu.VMEM((_N, 2 * _NH), jnp.float32),
        ],
    )(adj_2d, seq1_2d, seq2_2d, W_fc, b1r, a11, msk,
      W_bil.reshape(_NH, _NH), bbil11)

    logits = sc.T.reshape(1, 2 * _N)
    return logits + jnp.concatenate([samp_bias1, samp_bias2], axis=1)
